# baseline (speedup 1.0000x reference)
import functools

import jax
import jax.numpy as jnp
from jax import lax
from jax.experimental import pallas as pl
from jax.experimental.pallas import tpu as pltpu
from jax.experimental.pallas import tpu_sc as plsc

_DENSITY = 0.05
_TOP_IGNORE = 7
_B, _N = 128, 32768
_NC, _NS, _L = 2, 16, 16
_NW = _NC * _NS
_ROWS_PER_W = _B // _NW
_NCHUNK = _N // _L
_HSIZE = 1 << 15
_KLO = int(_N * _DENSITY) + _TOP_IGNORE
_KHI = _TOP_IGNORE
_MANT = 0x7FFFFFFF


def _keymap(v):
    u = lax.bitcast_convert_type(v, jnp.int32)
    return jnp.where(u >= 0, u, u ^ _MANT)


_UNROLL = 4


def _sc_body(x_hbm, out_hbm, xrow_a, xrow_b, hist, outs, sem_a, sem_b):
    wid = lax.axis_index("s") * _NC + lax.axis_index("c")
    base = wid * _ROWS_PER_W
    ones = jnp.ones((_L,), jnp.int32)
    zeros = jnp.zeros((_L,), jnp.int32)
    lanes = lax.iota(jnp.int32, _L)

    bufs = [(xrow_a, sem_a), (xrow_b, sem_b)]
    cps = [None, None]
    cps[0] = pltpu.async_copy(x_hbm.at[base], xrow_a, sem_a)

    for ri in range(_ROWS_PER_W):
        xrow, _ = bufs[ri % 2]
        cps[ri % 2].wait()
        if ri + 1 < _ROWS_PER_W:
            nbuf, nsem = bufs[(ri + 1) % 2]
            cps[(ri + 1) % 2] = pltpu.async_copy(
                x_hbm.at[base + ri + 1], nbuf, nsem)

        @plsc.parallel_loop(0, _HSIZE // _L, unroll=_UNROLL)
        def _clr(i):
            hist[pl.ds(i * _L, _L)] = zeros

        @plsc.parallel_loop(0, _NCHUNK, unroll=_UNROLL)
        def _pa(i):
            d = (_keymap(xrow[pl.ds(i * _L, _L)]) >> 17) + 16384
            plsc.addupdate_scatter(hist, [d], ones)

        def blk_scan(i, carry):
            r_cum, blk_a, above_a, blk_b, above_b = carry
            blk = 127 - i

            vs = [hist[pl.ds(blk * 256 + j * _L, _L)] for j in range(16)]
            while len(vs) > 1:
                vs = [a + b for a, b in zip(vs[::2], vs[1::2])]
            s = jnp.sum(vs[0])
            r_new = r_cum + s
            hit_a = (r_cum < _KLO) & (r_new >= _KLO)
            hit_b = (r_cum < _KHI) & (r_new >= _KHI)
            blk_a = jnp.where(hit_a, blk, blk_a)
            above_a = jnp.where(hit_a, r_cum, above_a)
            blk_b = jnp.where(hit_b, blk, blk_b)
            above_b = jnp.where(hit_b, r_cum, above_b)
            return r_new, blk_a, above_a, blk_b, above_b

        z = jnp.int32(0)
        _, blk_a, above_a, blk_b, above_b = lax.fori_loop(
            0, 128, blk_scan, (z, z, z, z, z))

        def locate(blk, above, r):
            def gscan(j, carry):
                r_cum, selg, sel_above = carry
                g = 15 - j
                v = hist[pl.ds(blk * 256 + g * _L, _L)]
                r_new = r_cum + jnp.sum(v)
                hit = (r_cum < r) & (r_new >= r)
                selg = jnp.where(hit, g, selg)
                sel_above = jnp.where(hit, r_cum, sel_above)
                return r_new, selg, sel_above

            _, selg, sel_above = lax.fori_loop(0, 16, gscan, (above, z, z))
            v = hist[pl.ds(blk * 256 + selg * _L, _L)]
            suffix = lax.rev(plsc.cumsum(lax.rev(v, (0,))), (0,))
            m = (sel_above + suffix) >= r
            lstar = plsc.all_reduce_population_count(m) - 1
            sel_suf = jnp.sum(jnp.where(lanes == lstar, suffix, 0))
            sel_h = jnp.sum(jnp.where(lanes == lstar, v, 0))
            rprime = r - (sel_above + sel_suf - sel_h)
            p = blk * 256 + selg * _L + lstar
            return p, jnp.broadcast_to(rprime, (_L,))

        p_a, rp_a = locate(blk_a, above_a, _KLO)
        p_b, rp_b = locate(blk_b, above_b, _KHI)

        cnt_b = plsc.load_gather(hist, [p_b])
        cnt_above = jnp.where(p_a == p_b, 0, cnt_b)
        cnt = plsc.load_gather(hist, [p_a]) + cnt_above

        @plsc.parallel_loop(0, _NCHUNK, unroll=_UNROLL, carry=zeros)
        def _pc(i, off):
            ks = _keymap(xrow[pl.ds(i * _L, _L)])
            d = (ks >> 17) + 16384
            m = (d == p_a) | (d == p_b)
            pos = jnp.where(
                m, plsc.cumsum(m.astype(jnp.int32)) - 1 + off, 0)
            plsc.store_scatter(hist, [pos], ks, mask=m)
            return off + plsc.all_reduce_population_count(m)

        del _pc

        def bisect(cnt_total, rp_splat, p_splat):
            nc = (lax.reduce_max(cnt_total, (0,)) + _L - 1) // _L
            kbase = (p_splat - 16384) * (1 << 17)

            def bitloop(bi, acc):
                bit = lax.shift_left(ones, jnp.broadcast_to(16 - bi, (_L,)))
                tv = kbase + (acc | bit)

                def cl(j, cv):
                    w = hist[pl.ds(j * _L, _L)]
                    mm = (w >= tv) & ((j * _L + lanes) < cnt_total)
                    return cv + plsc.all_reduce_population_count(mm)

                cv = lax.fori_loop(0, nc, cl, zeros)
                return jnp.where(cv >= rp_splat, acc | bit, acc)

            acc = lax.fori_loop(0, 17, bitloop, zeros)
            ksstar = kbase + acc
            bits = jnp.where(ksstar >= 0, ksstar, ksstar ^ _MANT)
            return lax.bitcast_convert_type(bits, jnp.float32)

        t_lo = bisect(cnt, rp_a + cnt_above, p_a)
        t_hi = bisect(cnt, rp_b, p_b)
        outs[pl.ds(ri * _L, _L)] = jnp.where(
            lanes == 0, t_lo, jnp.where(lanes == 1, t_hi, 0.0))

    pltpu.sync_copy(outs, out_hbm.at[wid])


def _sc_select(x):
    mesh = plsc.VectorSubcoreMesh(core_axis_name="c", subcore_axis_name="s")
    f = functools.partial(
        pl.kernel,
        mesh=mesh,
        out_type=jax.ShapeDtypeStruct((_NW, _ROWS_PER_W * _L), jnp.float32),
        scratch_types=[
            pltpu.VMEM((_N,), jnp.float32),
            pltpu.VMEM((_N,), jnp.float32),
            pltpu.VMEM((_HSIZE,), jnp.int32),
            pltpu.VMEM((_ROWS_PER_W * _L,), jnp.float32),
            pltpu.SemaphoreType.DMA,
            pltpu.SemaphoreType.DMA,
        ],
        compiler_params=pltpu.CompilerParams(needs_layout_passes=False),
    )(_sc_body)
    return f(x).reshape(_B, _L)


def _combine_body(x_ref, thr_ref, o_ref):
    x = x_ref[...]
    t_lo = thr_ref[:, 0:1]
    t_hi = thr_ref[:, 1:2]
    below = x <= t_hi
    inband = below & (x >= t_lo)
    cmean = jnp.mean(x, axis=0, keepdims=True)
    o_ref[...] = jnp.where(inband, x, 0.0) - cmean * below.astype(jnp.float32)


def kernel(x):
    n_rows, n = x.shape
    thr = _sc_select(x)
    nt = 8
    ct = n // nt
    return pl.pallas_call(
        _combine_body,
        grid=(nt,),
        in_specs=[
            pl.BlockSpec((n_rows, ct), lambda t: (0, t)),
            pl.BlockSpec((n_rows, _L), lambda t: (0, 0)),
        ],
        out_specs=pl.BlockSpec((n_rows, ct), lambda t: (0, t)),
        out_shape=jax.ShapeDtypeStruct((n_rows, n), jnp.float32),
    )(x, thr)


# speedup vs baseline: 1.0673x; 1.0673x over previous
import functools

import jax
import jax.numpy as jnp
from jax import lax
from jax.experimental import pallas as pl
from jax.experimental.pallas import tpu as pltpu
from jax.experimental.pallas import tpu_sc as plsc

_DENSITY = 0.05
_TOP_IGNORE = 7
_B, _N = 128, 32768
_NC, _NS, _L = 2, 16, 16
_NW = _NC * _NS
_ROWS_PER_W = _B // _NW
_NCHUNK = _N // _L
_HSIZE = 1 << 15
_KLO = int(_N * _DENSITY) + _TOP_IGNORE
_KHI = _TOP_IGNORE
_MANT = 0x7FFFFFFF


def _keymap(v):
    u = lax.bitcast_convert_type(v, jnp.int32)
    return jnp.where(u >= 0, u, u ^ _MANT)


_UNROLL = 8


def _sc_body(x_hbm, out_hbm, xrow_a, xrow_b, hist, outs, sem_a, sem_b):
    wid = lax.axis_index("s") * _NC + lax.axis_index("c")
    base = wid * _ROWS_PER_W
    ones = jnp.ones((_L,), jnp.int32)
    zeros = jnp.zeros((_L,), jnp.int32)
    lanes = lax.iota(jnp.int32, _L)

    bufs = [(xrow_a, sem_a), (xrow_b, sem_b)]
    cps = [None, None]
    cps[0] = pltpu.async_copy(x_hbm.at[base], xrow_a, sem_a)

    for ri in range(_ROWS_PER_W):
        xrow, _ = bufs[ri % 2]
        cps[ri % 2].wait()
        if ri + 1 < _ROWS_PER_W:
            nbuf, nsem = bufs[(ri + 1) % 2]
            cps[(ri + 1) % 2] = pltpu.async_copy(
                x_hbm.at[base + ri + 1], nbuf, nsem)

        @plsc.parallel_loop(0, _HSIZE // _L, unroll=_UNROLL)
        def _clr(i):
            hist[pl.ds(i * _L, _L)] = zeros

        @plsc.parallel_loop(0, _NCHUNK, unroll=_UNROLL)
        def _pa(i):
            d = (_keymap(xrow[pl.ds(i * _L, _L)]) >> 17) + 16384
            plsc.addupdate_scatter(hist, [d], ones)

        def blk_scan(i, carry):
            r_cum, blk_a, above_a, blk_b, above_b = carry
            blk = 127 - i

            vs = [hist[pl.ds(blk * 256 + j * _L, _L)] for j in range(16)]
            while len(vs) > 1:
                vs = [a + b for a, b in zip(vs[::2], vs[1::2])]
            s = jnp.sum(vs[0])
            r_new = r_cum + s
            hit_a = (r_cum < _KLO) & (r_new >= _KLO)
            hit_b = (r_cum < _KHI) & (r_new >= _KHI)
            blk_a = jnp.where(hit_a, blk, blk_a)
            above_a = jnp.where(hit_a, r_cum, above_a)
            blk_b = jnp.where(hit_b, blk, blk_b)
            above_b = jnp.where(hit_b, r_cum, above_b)
            return r_new, blk_a, above_a, blk_b, above_b

        z = jnp.int32(0)
        _, blk_a, above_a, blk_b, above_b = lax.fori_loop(
            0, 128, blk_scan, (z, z, z, z, z))

        def locate(blk, above, r):
            def gscan(j, carry):
                r_cum, selg, sel_above = carry
                g = 15 - j
                v = hist[pl.ds(blk * 256 + g * _L, _L)]
                r_new = r_cum + jnp.sum(v)
                hit = (r_cum < r) & (r_new >= r)
                selg = jnp.where(hit, g, selg)
                sel_above = jnp.where(hit, r_cum, sel_above)
                return r_new, selg, sel_above

            _, selg, sel_above = lax.fori_loop(0, 16, gscan, (above, z, z))
            v = hist[pl.ds(blk * 256 + selg * _L, _L)]
            suffix = lax.rev(plsc.cumsum(lax.rev(v, (0,))), (0,))
            m = (sel_above + suffix) >= r
            lstar = plsc.all_reduce_population_count(m) - 1
            sel_suf = jnp.sum(jnp.where(lanes == lstar, suffix, 0))
            sel_h = jnp.sum(jnp.where(lanes == lstar, v, 0))
            rprime = r - (sel_above + sel_suf - sel_h)
            p = blk * 256 + selg * _L + lstar
            return p, jnp.broadcast_to(rprime, (_L,))

        p_a, rp_a = locate(blk_a, above_a, _KLO)
        p_b, rp_b = locate(blk_b, above_b, _KHI)

        cnt_b = plsc.load_gather(hist, [p_b])
        cnt_above = jnp.where(p_a == p_b, 0, cnt_b)
        cnt = plsc.load_gather(hist, [p_a]) + cnt_above

        @plsc.parallel_loop(0, _NCHUNK, unroll=_UNROLL, carry=zeros)
        def _pc(i, off):
            ks = _keymap(xrow[pl.ds(i * _L, _L)])
            d = (ks >> 17) + 16384
            m = (d == p_a) | (d == p_b)
            pos = jnp.where(
                m, plsc.cumsum(m.astype(jnp.int32)) - 1 + off, 0)
            plsc.store_scatter(hist, [pos], ks, mask=m)
            return off + plsc.all_reduce_population_count(m)

        del _pc

        def bisect(cnt_total, rp_splat, p_splat):
            nc = (lax.reduce_max(cnt_total, (0,)) + _L - 1) // _L
            kbase = (p_splat - 16384) * (1 << 17)

            def bitloop(bi, acc):
                bit = lax.shift_left(ones, jnp.broadcast_to(16 - bi, (_L,)))
                tv = kbase + (acc | bit)

                def cl(j, cv):
                    w = hist[pl.ds(j * _L, _L)]
                    mm = (w >= tv) & ((j * _L + lanes) < cnt_total)
                    return cv + plsc.all_reduce_population_count(mm)

                cv = lax.fori_loop(0, nc, cl, zeros)
                return jnp.where(cv >= rp_splat, acc | bit, acc)

            acc = lax.fori_loop(0, 17, bitloop, zeros)
            ksstar = kbase + acc
            bits = jnp.where(ksstar >= 0, ksstar, ksstar ^ _MANT)
            return lax.bitcast_convert_type(bits, jnp.float32)

        t_lo = bisect(cnt, rp_a + cnt_above, p_a)
        t_hi = bisect(cnt, rp_b, p_b)
        outs[pl.ds(ri * _L, _L)] = jnp.where(
            lanes == 0, t_lo, jnp.where(lanes == 1, t_hi, 0.0))

    pltpu.sync_copy(outs, out_hbm.at[wid])


def _sc_select(x):
    mesh = plsc.VectorSubcoreMesh(core_axis_name="c", subcore_axis_name="s")
    f = functools.partial(
        pl.kernel,
        mesh=mesh,
        out_type=jax.ShapeDtypeStruct((_NW, _ROWS_PER_W * _L), jnp.float32),
        scratch_types=[
            pltpu.VMEM((_N,), jnp.float32),
            pltpu.VMEM((_N,), jnp.float32),
            pltpu.VMEM((_HSIZE,), jnp.int32),
            pltpu.VMEM((_ROWS_PER_W * _L,), jnp.float32),
            pltpu.SemaphoreType.DMA,
            pltpu.SemaphoreType.DMA,
        ],
        compiler_params=pltpu.CompilerParams(needs_layout_passes=False),
    )(_sc_body)
    return f(x).reshape(_B, _L)


def _combine_body(x_ref, thr_ref, o_ref):
    x = x_ref[...]
    t_lo = thr_ref[:, 0:1]
    t_hi = thr_ref[:, 1:2]
    below = x <= t_hi
    inband = below & (x >= t_lo)
    cmean = jnp.mean(x, axis=0, keepdims=True)
    o_ref[...] = jnp.where(inband, x, 0.0) - cmean * below.astype(jnp.float32)


def kernel(x):
    n_rows, n = x.shape
    thr = _sc_select(x)
    nt = 8
    ct = n // nt
    return pl.pallas_call(
        _combine_body,
        grid=(nt,),
        in_specs=[
            pl.BlockSpec((n_rows, ct), lambda t: (0, t)),
            pl.BlockSpec((n_rows, _L), lambda t: (0, 0)),
        ],
        out_specs=pl.BlockSpec((n_rows, ct), lambda t: (0, t)),
        out_shape=jax.ShapeDtypeStruct((n_rows, n), jnp.float32),
    )(x, thr)
